# Initial kernel scaffold; baseline (speedup 1.0000x reference)
import jax
import jax.numpy as jnp
from jax.experimental import pallas as pl


def kernel(features, edge_index, W1, b1, W2, b2, bn_gamma, bn_beta, fc1_W, fc1_b, fc2_W, fc2_b):
    raise NotImplementedError("write your pallas kernel here")


# speedup vs baseline: 5.9824x; 5.9824x over previous
import functools

import jax
import jax.numpy as jnp
from jax import lax
from jax.experimental import pallas as pl
from jax.experimental.pallas import tpu as pltpu
from jax.experimental.pallas import tpu_sc as plsc

N_NODES = 10000
N_EDGES = 320000
D = 128
N_CLASSES = 40
N_LAYERS = 3
BN_EPS = 1e-5

NC = 2
NS = 16
NW = NC * NS
E_W = N_EDGES // NW
CH = 80
NCH = E_W // CH
INIT_TILES = 10
R_T = N_NODES // INIT_TILES

_sc_mesh = plsc.VectorSubcoreMesh(core_axis_name="c", subcore_axis_name="s")


@functools.partial(
    pl.kernel,
    out_type=jax.ShapeDtypeStruct((NC * N_NODES, D), jnp.float32),
    mesh=_sc_mesh,
    scratch_types=[
        pltpu.VMEM((NCH, CH), jnp.int32),
        pltpu.VMEM((NCH, CH), jnp.int32),
        pltpu.VMEM((CH, D), jnp.float32),
        pltpu.VMEM_SHARED((N_NODES, D), jnp.float32),
        pltpu.SemaphoreType.DMA,
    ],
)
def _segsum_sc(h_hbm, src_hbm, dst_hbm, zeros_hbm, out_hbm,
               src_v, dst_v, rows_v, acc_sh, sem):
    cid = lax.axis_index("c")
    sid = lax.axis_index("s")
    wid = sid * NC + cid
    pltpu.sync_copy(src_hbm.at[wid], src_v)
    pltpu.sync_copy(dst_hbm.at[wid], dst_v)
    @pl.when(sid < INIT_TILES)
    def _():
        pltpu.sync_copy(zeros_hbm.at[pl.ds(sid * R_T, R_T)],
                        acc_sh.at[pl.ds(sid * R_T, R_T)])

    plsc.subcore_barrier()

    def body(j, carry):
        pltpu.async_copy(h_hbm.at[src_v.at[j]], rows_v, sem).wait()
        pltpu.sync_copy(rows_v, acc_sh.at[dst_v.at[j]], add=True)
        return carry

    lax.fori_loop(0, NCH, body, 0)
    plsc.subcore_barrier()

    @pl.when(sid < INIT_TILES)
    def _():
        pltpu.sync_copy(acc_sh.at[pl.ds(sid * R_T, R_T)],
                        out_hbm.at[pl.ds(cid * N_NODES + sid * R_T, R_T)])


ROWS_B = 1000
N_BLK = N_NODES // ROWS_B


def _mlp_body(h_ref, p0_ref, p1_ref, w1_ref, b1_ref, w2_ref, b2_ref,
              y_ref, sum_ref, sq_ref):
    x = h_ref[...] + p0_ref[...] + p1_ref[...]
    t = jnp.maximum(
        jnp.dot(x, w1_ref[...], preferred_element_type=jnp.float32)
        + b1_ref[...], 0.0)
    y = jnp.maximum(
        jnp.dot(t, w2_ref[...], preferred_element_type=jnp.float32)
        + b2_ref[...], 0.0)
    y_ref[...] = y
    i = pl.program_id(0)

    @pl.when(i == 0)
    def _():
        sum_ref[...] = jnp.zeros_like(sum_ref)
        sq_ref[...] = jnp.zeros_like(sq_ref)

    sum_ref[...] += jnp.sum(y, axis=0, keepdims=True)
    sq_ref[...] += jnp.sum(y * y, axis=0, keepdims=True)


_mlp_call = pl.pallas_call(
    _mlp_body,
    grid=(N_BLK,),
    in_specs=[
        pl.BlockSpec((ROWS_B, D), lambda i: (i, 0)),
        pl.BlockSpec((ROWS_B, D), lambda i: (i, 0)),
        pl.BlockSpec((ROWS_B, D), lambda i: (i, 0)),
        pl.BlockSpec((D, D), lambda i: (0, 0)),
        pl.BlockSpec((1, D), lambda i: (0, 0)),
        pl.BlockSpec((D, D), lambda i: (0, 0)),
        pl.BlockSpec((1, D), lambda i: (0, 0)),
    ],
    out_specs=[
        pl.BlockSpec((ROWS_B, D), lambda i: (i, 0)),
        pl.BlockSpec((1, D), lambda i: (0, 0)),
        pl.BlockSpec((1, D), lambda i: (0, 0)),
    ],
    out_shape=[
        jax.ShapeDtypeStruct((N_NODES, D), jnp.float32),
        jax.ShapeDtypeStruct((1, D), jnp.float32),
        jax.ShapeDtypeStruct((1, D), jnp.float32),
    ],
)


def _bn_body(y_ref, sum_ref, sq_ref, g_ref, be_ref, out_ref):
    mean = sum_ref[...] * (1.0 / N_NODES)
    var = sq_ref[...] * (1.0 / N_NODES) - mean * mean
    a = lax.rsqrt(var + BN_EPS) * g_ref[...]
    b = be_ref[...] - mean * a
    out_ref[...] = y_ref[...] * a + b


_bn_call = pl.pallas_call(
    _bn_body,
    grid=(N_BLK,),
    in_specs=[
        pl.BlockSpec((ROWS_B, D), lambda i: (i, 0)),
        pl.BlockSpec((1, D), lambda i: (0, 0)),
        pl.BlockSpec((1, D), lambda i: (0, 0)),
        pl.BlockSpec((1, D), lambda i: (0, 0)),
        pl.BlockSpec((1, D), lambda i: (0, 0)),
    ],
    out_specs=pl.BlockSpec((ROWS_B, D), lambda i: (i, 0)),
    out_shape=jax.ShapeDtypeStruct((N_NODES, D), jnp.float32),
)


def _head_body(h_ref, w1_ref, b1_ref, w2_ref, b2_ref, out_ref):
    t = jnp.maximum(
        jnp.dot(h_ref[...], w1_ref[...], preferred_element_type=jnp.float32)
        + b1_ref[...], 0.0)
    logits = (jnp.dot(t, w2_ref[...], preferred_element_type=jnp.float32)
              + b2_ref[...])
    m = jnp.max(logits, axis=1, keepdims=True)
    z = logits - m
    lse = jnp.log(jnp.sum(jnp.exp(z), axis=1, keepdims=True))
    out_ref[...] = z - lse


_head_call = pl.pallas_call(
    _head_body,
    grid=(N_BLK,),
    in_specs=[
        pl.BlockSpec((ROWS_B, D), lambda i: (i, 0)),
        pl.BlockSpec((D, D), lambda i: (0, 0)),
        pl.BlockSpec((1, D), lambda i: (0, 0)),
        pl.BlockSpec((D, N_CLASSES), lambda i: (0, 0)),
        pl.BlockSpec((1, N_CLASSES), lambda i: (0, 0)),
    ],
    out_specs=pl.BlockSpec((ROWS_B, N_CLASSES), lambda i: (i, 0)),
    out_shape=jax.ShapeDtypeStruct((N_NODES, N_CLASSES), jnp.float32),
)


def kernel(features, edge_index, W1, b1, W2, b2, bn_gamma, bn_beta,
           fc1_W, fc1_b, fc2_W, fc2_b):
    src = edge_index[0].astype(jnp.int32).reshape(NW, NCH, CH)
    dst = edge_index[1].astype(jnp.int32).reshape(NW, NCH, CH)
    zeros = jnp.zeros((N_NODES, D), jnp.float32)
    b1r = b1.reshape(N_LAYERS, 1, D)
    b2r = b2.reshape(N_LAYERS, 1, D)
    gr = bn_gamma.reshape(N_LAYERS, 1, D)
    ber = bn_beta.reshape(N_LAYERS, 1, D)

    h = features
    for i in range(N_LAYERS):
        parts = _segsum_sc(h, src, dst, zeros)
        y, s, sq = _mlp_call(h, parts[:N_NODES], parts[N_NODES:],
                             W1[i], b1r[i], W2[i], b2r[i])
        h = _bn_call(y, s, sq, gr[i], ber[i])
    return _head_call(h, fc1_W, fc1_b.reshape(1, D), fc2_W,
                      fc2_b.reshape(1, N_CLASSES))
